# Initial kernel scaffold; baseline (speedup 1.0000x reference)
import jax
import jax.numpy as jnp
from jax.experimental import pallas as pl


def kernel(pd, gt):
    raise NotImplementedError("write your pallas kernel here")


# speedup vs baseline: 48.6240x; 48.6240x over previous
import functools

import jax
import jax.numpy as jnp
from jax import lax
from jax.experimental import pallas as pl
from jax.experimental.pallas import tpu as pltpu
from jax.experimental.pallas import tpu_sc as plsc

B = 4
C = 19
HW = 512 * 512
P = B * HW
K = 512
NLANES = 16
NCORES = 2
NSUB = 16
NW = NCORES * NSUB
PER_W = P // NW
WPB = HW // PER_W
UNROLL = 4


def _sc_hist_kernel(pd_hbm, gt_hbm, out_hbm, gts, ps, hist, red):
    wid = lax.axis_index("s") * NCORES + lax.axis_index("c")
    b = wid // WPB
    r = (wid % WPB) * PER_W

    pltpu.sync_copy(gt_hbm.at[pl.ds(wid * PER_W, PER_W)], gts)
    lane_base = lax.iota(jnp.int32, NLANES) * K

    for c in range(C):
        off = (b * C + c) * HW + r
        pltpu.sync_copy(pd_hbm.at[pl.ds(off, PER_W)], ps)

        def zero_body(j, carry):
            hist[pl.ds(j * NLANES, NLANES)] = jnp.zeros((NLANES,), jnp.int32)
            return carry

        lax.fori_loop(0, K, zero_body, 0)

        def acc_body(i, carry):
            for u in range(UNROLL):
                base = (i * UNROLL + u) * NLANES
                p = ps[pl.ds(base, NLANES)]
                g = gts[pl.ds(base, NLANES)]
                fg = g == c
                e = jnp.where(fg, 1.0 - p, p)
                bin_ = jnp.minimum((e * K).astype(jnp.int32), K - 1)
                val = jnp.where(fg, 65537, 1).astype(jnp.int32)
                plsc.addupdate_scatter(hist, [lane_base + bin_], val)
            return carry

        lax.fori_loop(0, PER_W // (NLANES * UNROLL), acc_body, 0)

        def red_body(j, carry):
            acc = hist[pl.ds(j * NLANES, NLANES)]
            for l in range(1, NLANES):
                acc = acc + hist[pl.ds(l * K + j * NLANES, NLANES)]
            red[pl.ds(j * NLANES, NLANES)] = acc
            return carry

        lax.fori_loop(0, K // NLANES, red_body, 0)
        pltpu.sync_copy(red, out_hbm.at[pl.ds((wid * C + c) * K, K)])


_sc_hist = functools.partial(
    pl.kernel,
    mesh=plsc.VectorSubcoreMesh(core_axis_name="c", subcore_axis_name="s"),
    out_type=jax.ShapeDtypeStruct((NW * C * K,), jnp.int32),
    compiler_params=pltpu.CompilerParams(needs_layout_passes=False),
    scratch_types=[
        pltpu.VMEM((PER_W,), jnp.int32),
        pltpu.VMEM((PER_W,), jnp.float32),
        pltpu.VMEM((NLANES * K,), jnp.int32),
        pltpu.VMEM((K,), jnp.int32),
    ],
)(_sc_hist_kernel)


def _finalize_kernel(h_ref, o_ref):
    h = h_ref[...]
    n = (h & 0xFFFF).astype(jnp.float32)
    f = lax.shift_right_logical(h, 16).astype(jnp.float32)
    n = jnp.sum(n, axis=0)
    f = jnp.sum(f, axis=0)

    ii = lax.broadcasted_iota(jnp.int32, (K, K), 0)
    jj = lax.broadcasted_iota(jnp.int32, (K, K), 1)
    tri = (ii >= jj).astype(jnp.float32)
    r_suf = jnp.dot(n, tri, preferred_element_type=jnp.float32)
    cf_suf = jnp.dot(f, tri, preferred_element_type=jnp.float32)

    s = cf_suf[:, 0:1]
    u = s + r_suf - cf_suf
    jac = jnp.where(u > 0, 1.0 - (s - cf_suf) / jnp.maximum(u, 1.0), 0.0)
    loss_c = jnp.sum(jac, axis=1) / K - 0.5 * jac[:, 0] / K
    o_ref[...] = jnp.reshape(jnp.sum(loss_c) / C, (1, 1))


def kernel(pd, gt):
    pd_flat = pd.reshape(-1)
    gt_flat = gt.reshape(-1).astype(jnp.int32)
    hist = _sc_hist(pd_flat, gt_flat)
    out = pl.pallas_call(
        _finalize_kernel,
        out_shape=jax.ShapeDtypeStruct((1, 1), jnp.float32),
    )(hist.reshape(NW, C, K))
    return out[0, 0]


# speedup vs baseline: 152.4243x; 3.1348x over previous
import functools

import jax
import jax.numpy as jnp
from jax import lax
from jax.experimental import pallas as pl
from jax.experimental.pallas import tpu as pltpu
from jax.experimental.pallas import tpu_sc as plsc

B = 4
C = 19
HW = 512 * 512
P = B * HW
K = 512
NLANES = 16
NCORES = 2
NSUB = 16
NW = NCORES * NSUB
PER_W = P // NW
WPB = HW // PER_W
UNROLL = 4


def _sc_hist_kernel(pd_hbm, gt_hbm, out_hbm, gts, ps, hist, red):
    wid = lax.axis_index("s") * NCORES + lax.axis_index("c")
    b = wid // WPB
    r = (wid % WPB) * PER_W

    pltpu.sync_copy(gt_hbm.at[pl.ds(wid * PER_W, PER_W)], gts)
    lane_base = lax.iota(jnp.int32, NLANES) * K

    for c in range(C):
        off = (b * C + c) * HW + r
        pltpu.sync_copy(pd_hbm.at[pl.ds(off, PER_W)], ps)

        @plsc.parallel_loop(0, NLANES * K // NLANES, unroll=8)
        def zero_body(j):
            hist[pl.ds(j * NLANES, NLANES)] = jnp.zeros((NLANES,), jnp.int32)

        @plsc.parallel_loop(0, PER_W // NLANES, unroll=8)
        def acc_body(i):
            base = i * NLANES
            p = ps[pl.ds(base, NLANES)]
            g = gts[pl.ds(base, NLANES)]
            fg = g == c
            e = jnp.where(fg, 1.0 - p, p)
            bin_ = jnp.minimum((e * K).astype(jnp.int32), K - 1)
            val = jnp.where(fg, 65537, 1).astype(jnp.int32)
            plsc.addupdate_scatter(hist, [lane_base + bin_], val)

        @plsc.parallel_loop(0, K // NLANES, unroll=2)
        def red_body(j):
            acc = hist[pl.ds(j * NLANES, NLANES)]
            for l in range(1, NLANES):
                acc = acc + hist[pl.ds(l * K + j * NLANES, NLANES)]
            red[pl.ds(j * NLANES, NLANES)] = acc
        pltpu.sync_copy(red, out_hbm.at[pl.ds((wid * C + c) * K, K)])


_sc_hist = functools.partial(
    pl.kernel,
    mesh=plsc.VectorSubcoreMesh(core_axis_name="c", subcore_axis_name="s"),
    out_type=jax.ShapeDtypeStruct((NW * C * K,), jnp.int32),
    compiler_params=pltpu.CompilerParams(needs_layout_passes=False),
    scratch_types=[
        pltpu.VMEM((PER_W,), jnp.int32),
        pltpu.VMEM((PER_W,), jnp.float32),
        pltpu.VMEM((NLANES * K,), jnp.int32),
        pltpu.VMEM((K,), jnp.int32),
    ],
)(_sc_hist_kernel)


def _finalize_kernel(h_ref, o_ref):
    h = h_ref[...]
    n = (h & 0xFFFF).astype(jnp.float32)
    f = lax.shift_right_logical(h, 16).astype(jnp.float32)
    n = jnp.sum(n, axis=0)
    f = jnp.sum(f, axis=0)

    ii = lax.broadcasted_iota(jnp.int32, (K, K), 0)
    jj = lax.broadcasted_iota(jnp.int32, (K, K), 1)
    tri = (ii >= jj).astype(jnp.float32)
    r_suf = jnp.dot(n, tri, preferred_element_type=jnp.float32)
    cf_suf = jnp.dot(f, tri, preferred_element_type=jnp.float32)

    s = cf_suf[:, 0:1]
    u = s + r_suf - cf_suf
    jac = jnp.where(u > 0, 1.0 - (s - cf_suf) / jnp.maximum(u, 1.0), 0.0)
    loss_c = jnp.sum(jac, axis=1) / K - 0.5 * jac[:, 0] / K
    o_ref[...] = jnp.reshape(jnp.sum(loss_c) / C, (1, 1))


def kernel(pd, gt):
    pd_flat = pd.reshape(-1)
    gt_flat = gt.reshape(-1).astype(jnp.int32)
    hist = _sc_hist(pd_flat, gt_flat)
    out = pl.pallas_call(
        _finalize_kernel,
        out_shape=jax.ShapeDtypeStruct((1, 1), jnp.float32),
    )(hist.reshape(NW, C, K))
    return out[0, 0]
